# baseline (speedup 1.0000x reference)
import functools

import jax
import jax.numpy as jnp
from jax import lax
from jax.experimental import pallas as pl
from jax.experimental.pallas import tpu as pltpu
from jax.experimental.pallas import tpu_sc as plsc

_N = 10000
_E = 320000
_D = 128
_NREL = 3
_NHEADS = 4

_C = 128
_NW = 32
_CPT = 80
_HALF = _CPT // 2
_NCHUNK = _NW * _CPT
_NREAL = _E // _C
_NB = 2
_NG = _HALF // _NB
_RPT = 632
_RPT_LAST = _N - 15 * _RPT
_ZROW = _NREL * _N

_BN = 1000


def _dense_body(h_ref, w_ref, q_ref, out_ref):
    hb = h_ref[...]
    for r in range(_NREL):
        acc = None
        qr = q_ref[r]
        for hd in range(_NHEADS):
            wm = w_ref[r, hd]
            hp = lax.dot_general(hb, wm, (((1,), (1,)), ((), ())),
                                 preferred_element_type=jnp.float32)
            qv = qr[hd:hd + 1, :]
            g = jnp.sum(hp * qv, axis=1, keepdims=True)
            gate = jax.nn.sigmoid(g)
            term = hp * gate
            acc = term if acc is None else acc + term
        out_ref[r] = acc * (1.0 / _NHEADS)


def _dense_call(h, W, q_pad):
    grid = _N // _BN
    return pl.pallas_call(
        _dense_body,
        grid=(grid,),
        in_specs=[
            pl.BlockSpec((_BN, _D), lambda i: (i, 0)),
            pl.BlockSpec((_NREL, _NHEADS, _D, _D), lambda i: (0, 0, 0, 0)),
            pl.BlockSpec((_NREL, 8, _D), lambda i: (0, 0, 0)),
        ],
        out_specs=pl.BlockSpec((_NREL, _BN, _D), lambda i: (0, i, 0)),
        out_shape=jax.ShapeDtypeStruct((_NREL, _N, _D), jnp.float32),
    )(h, W, q_pad)


def _sc_body(xbar_hbm, srcs_hbm, dsts_hbm, zeros_hbm, out_hbm,
             acc, src_slab, dst_slab, b0, b1, s0, s1):
    c = lax.axis_index("c")
    s = lax.axis_index("s")
    wid = s * 2 + c
    base = s * _RPT
    bufs = (b0, b1)
    sems = (s0, s1)

    def gather_start(j, b):
        pltpu.async_copy(xbar_hbm.at[src_slab.at[j]], bufs[b], sems[b])

    def gather_wait(b):
        pltpu.make_async_copy(xbar_hbm.at[pl.ds(0, _C)], bufs[b],
                              sems[b]).wait()

    for r in range(_NREL):
        @pl.when(s < 15)
        def _():
            pltpu.sync_copy(zeros_hbm, acc.at[pl.ds(base, _RPT)])

        @pl.when(s == 15)
        def _():
            pltpu.sync_copy(zeros_hbm.at[pl.ds(0, _RPT_LAST)],
                            acc.at[pl.ds(base, _RPT_LAST)])

        plsc.subcore_barrier()

        for half in range(2):
            tbase = r * _NCHUNK + wid * _CPT + half * _HALF
            pltpu.sync_copy(srcs_hbm.at[pl.ds(tbase, _HALF)], src_slab)
            pltpu.sync_copy(dsts_hbm.at[pl.ds(tbase, _HALF)], dst_slab)

            def outer(g, carry):
                for b in range(_NB):
                    j = g * _NB + b
                    pltpu.async_copy(xbar_hbm.at[src_slab.at[j]], bufs[b],
                                     sems[b]).wait()
                    pltpu.sync_copy(bufs[b], acc.at[dst_slab.at[j]],
                                    add=True)
                return carry

            lax.fori_loop(0, _NG, outer, 0)

        plsc.subcore_barrier()

        out_off = (c * _NREL + r) * _N + base

        @pl.when(s < 15)
        def _():
            pltpu.sync_copy(acc.at[pl.ds(base, _RPT)],
                            out_hbm.at[pl.ds(out_off, _RPT)])

        @pl.when(s == 15)
        def _():
            pltpu.sync_copy(acc.at[pl.ds(base, _RPT_LAST)],
                            out_hbm.at[pl.ds(out_off, _RPT_LAST)])

        plsc.subcore_barrier()


def _sc_call(xbar_flat, srcs, dsts, zeros_hbm):
    mesh = plsc.VectorSubcoreMesh(core_axis_name="c", subcore_axis_name="s")
    return pl.kernel(
        _sc_body,
        out_type=jax.ShapeDtypeStruct((2 * _NREL * _N, _D), jnp.float32),
        mesh=mesh,
        scratch_types=[
            pltpu.VMEM_SHARED((_N, _D), jnp.float32),
            pltpu.VMEM((_HALF, _C), jnp.int32),
            pltpu.VMEM((_HALF, _C), jnp.int32),
            pltpu.VMEM((_C, _D), jnp.float32),
            pltpu.VMEM((_C, _D), jnp.float32),
            pltpu.SemaphoreType.DMA,
            pltpu.SemaphoreType.DMA,
        ],
    )(xbar_flat, srcs, dsts, zeros_hbm)


def _fuse_body(p_ref, rv_ref, bias_ref, h_ref, attn_ref):
    m0 = p_ref[0] + p_ref[3]
    m1 = p_ref[1] + p_ref[4]
    m2 = p_ref[2] + p_ref[5]
    rv = rv_ref[...]
    l0 = jnp.sum(m0 * rv[0:1, :], axis=1, keepdims=True) + bias_ref[0]
    l1 = jnp.sum(m1 * rv[1:2, :], axis=1, keepdims=True) + bias_ref[1]
    l2 = jnp.sum(m2 * rv[2:3, :], axis=1, keepdims=True) + bias_ref[2]
    mx = jnp.maximum(jnp.maximum(l0, l1), l2)
    e0 = jnp.exp(l0 - mx)
    e1 = jnp.exp(l1 - mx)
    e2 = jnp.exp(l2 - mx)
    inv = 1.0 / (e0 + e1 + e2)
    a0 = e0 * inv
    a1 = e1 * inv
    a2 = e2 * inv
    h_ref[...] = jnp.maximum(a0 * m0 + a1 * m1 + a2 * m2, 0.0)
    col = lax.broadcasted_iota(jnp.int32, (1, _D), 1)
    attn_ref[0] = jnp.where(
        col == 0, jnp.sum(a0),
        jnp.where(col == 1, jnp.sum(a1),
                  jnp.where(col == 2, jnp.sum(a2), 0.0)))


def _fuse_call(partials, rv_pad, relation_bias):
    grid = _N // _BN
    return pl.pallas_call(
        _fuse_body,
        grid=(grid,),
        in_specs=[
            pl.BlockSpec((2 * _NREL, _BN, _D), lambda i: (0, i, 0)),
            pl.BlockSpec((8, _D), lambda i: (0, 0)),
            pl.BlockSpec(memory_space=pltpu.SMEM),
        ],
        out_specs=[
            pl.BlockSpec((_BN, _D), lambda i: (i, 0)),
            pl.BlockSpec((1, 1, _D), lambda i: (i, 0, 0)),
        ],
        out_shape=[
            jax.ShapeDtypeStruct((_N, _D), jnp.float32),
            jax.ShapeDtypeStruct((grid, 1, _D), jnp.float32),
        ],
    )(partials, rv_pad, relation_bias)


def kernel(h, edge_index_r0, edge_index_r1, edge_index_r2,
           relation_bias, W, node_query, rel_vector):
    q_pad = jnp.zeros((_NREL, 8, _D), jnp.float32).at[:, :_NHEADS].set(node_query)
    rv_pad = jnp.zeros((8, _D), jnp.float32).at[:_NREL].set(rel_vector)

    n_pad = _NCHUNK - _NREAL
    src_all = jnp.stack([edge_index_r0[1], edge_index_r1[1], edge_index_r2[1]])
    src_all = src_all + (jnp.arange(_NREL, dtype=jnp.int32) * _N)[:, None]
    src_pad = jnp.full((_NREL, n_pad, _C), _ZROW, jnp.int32)
    srcs = jnp.concatenate(
        [src_all.reshape(_NREL, _NREAL, _C), src_pad], axis=1)
    srcs = srcs.reshape(_NREL * _NCHUNK, _C)
    dsts = jnp.stack([edge_index_r0[0], edge_index_r1[0], edge_index_r2[0]])
    dst_pad = jnp.arange(n_pad * _C, dtype=jnp.int32) * 61 % _N
    dst_pad = jnp.broadcast_to(dst_pad.reshape(1, n_pad, _C),
                               (_NREL, n_pad, _C))
    dsts = jnp.concatenate(
        [dsts.reshape(_NREL, _NREAL, _C), dst_pad], axis=1)
    dsts = dsts.reshape(_NREL * _NCHUNK, _C)
    zeros_hbm = jnp.zeros((_RPT, _D), jnp.float32)

    xbar = _dense_call(h, W, q_pad)
    xbar_flat = jnp.concatenate(
        [xbar.reshape(_NREL * _N, _D), jnp.zeros((8, _D), jnp.float32)])
    part = _sc_call(xbar_flat, srcs, dsts, zeros_hbm)
    h_next, attn_p = _fuse_call(part.reshape(2 * _NREL, _N, _D),
                                rv_pad, relation_bias)
    rel_attn_global = attn_p[:, 0, :_NREL].sum(axis=0) * (1.0 / _N)
    return (h_next, rel_attn_global)


# speedup vs baseline: 3.6719x; 3.6719x over previous
import functools

import jax
import jax.numpy as jnp
from jax import lax
from jax.experimental import pallas as pl
from jax.experimental.pallas import tpu as pltpu
from jax.experimental.pallas import tpu_sc as plsc

_N = 10000
_E = 320000
_D = 128
_NREL = 3
_NHEADS = 4

_C = 128
_NW = 32
_CPT = 80
_HALF = _CPT // 2
_NCHUNK = _NW * _CPT
_NREAL = _E // _C
_NB = 2
_NG = _HALF // _NB
_RPT = 632
_RPT_LAST = _N - 15 * _RPT
_ZROW = _NREL * _N

_BN = 1000


def _dense_body(h_ref, w_ref, q_ref, out_ref):
    hb = h_ref[...]
    for r in range(_NREL):
        acc = None
        qr = q_ref[r]
        for hd in range(_NHEADS):
            wm = w_ref[r, hd]
            hp = lax.dot_general(hb, wm, (((1,), (1,)), ((), ())),
                                 preferred_element_type=jnp.float32)
            qv = qr[hd:hd + 1, :]
            g = jnp.sum(hp * qv, axis=1, keepdims=True)
            gate = jax.nn.sigmoid(g)
            term = hp * gate
            acc = term if acc is None else acc + term
        out_ref[r] = acc * (1.0 / _NHEADS)


def _dense_call(h, W, q_pad):
    grid = _N // _BN
    return pl.pallas_call(
        _dense_body,
        grid=(grid,),
        in_specs=[
            pl.BlockSpec((_BN, _D), lambda i: (i, 0)),
            pl.BlockSpec((_NREL, _NHEADS, _D, _D), lambda i: (0, 0, 0, 0)),
            pl.BlockSpec((_NREL, 8, _D), lambda i: (0, 0, 0)),
        ],
        out_specs=pl.BlockSpec((_NREL, _BN, _D), lambda i: (0, i, 0)),
        out_shape=jax.ShapeDtypeStruct((_NREL, _N, _D), jnp.float32),
    )(h, W, q_pad)


def _sc_body(xbar_hbm, srcs_hbm, dsts_hbm, zeros_hbm, out_hbm,
             acc, si0, si1, di0, di1, rb0, rb1, mi0, mi1, mg0, mg1):
    c = lax.axis_index("c")
    s = lax.axis_index("s")
    wid = s * 2 + c
    base = s * _RPT
    sidx = (si0, si1)
    didx = (di0, di1)
    rows = (rb0, rb1)
    sem_i = (mi0, mi1)
    sem_g = (mg0, mg1)

    def idx_start(cid, k):
        pltpu.async_copy(srcs_hbm.at[cid], sidx[k], sem_i[k])
        pltpu.async_copy(dsts_hbm.at[cid], didx[k], sem_i[k])

    def idx_wait(k):
        pltpu.make_async_copy(srcs_hbm.at[0], sidx[k], sem_i[k]).wait()
        pltpu.make_async_copy(srcs_hbm.at[0], didx[k], sem_i[k]).wait()

    def gather_start(k):
        pltpu.async_copy(xbar_hbm.at[sidx[k]], rows[k], sem_g[k])

    def gather_wait(k):
        pltpu.make_async_copy(xbar_hbm.at[pl.ds(0, _C)], rows[k],
                              sem_g[k]).wait()

    for r in range(_NREL):
        @pl.when(s < 15)
        def _():
            pltpu.sync_copy(zeros_hbm, acc.at[pl.ds(base, _RPT)])

        @pl.when(s == 15)
        def _():
            pltpu.sync_copy(zeros_hbm.at[pl.ds(0, _RPT_LAST)],
                            acc.at[pl.ds(base, _RPT_LAST)])

        plsc.subcore_barrier()

        tbase = r * _NCHUNK + wid * _CPT
        idx_start(tbase, 0)
        idx_start(tbase + 1, 1)
        idx_wait(0)
        gather_start(0)

        def outer(g, carry):
            for b in range(_NB):
                j = g * _NB + b
                gather_wait(b)

                @pl.when(j < _CPT - 1)
                def _():
                    idx_wait(1 - b)
                    gather_start(1 - b)

                pltpu.sync_copy(rows[b], acc.at[didx[b]], add=True)

                @pl.when(j < _CPT - 2)
                def _():
                    idx_start(tbase + j + 2, b)
            return carry

        lax.fori_loop(0, _CPT // _NB, outer, 0)
        plsc.subcore_barrier()

        out_off = (c * _NREL + r) * _N + base

        @pl.when(s < 15)
        def _():
            pltpu.sync_copy(acc.at[pl.ds(base, _RPT)],
                            out_hbm.at[pl.ds(out_off, _RPT)])

        @pl.when(s == 15)
        def _():
            pltpu.sync_copy(acc.at[pl.ds(base, _RPT_LAST)],
                            out_hbm.at[pl.ds(out_off, _RPT_LAST)])

        plsc.subcore_barrier()


def _sc_call(xbar_flat, srcs, dsts, zeros_hbm):
    mesh = plsc.VectorSubcoreMesh(core_axis_name="c", subcore_axis_name="s")
    return pl.kernel(
        _sc_body,
        out_type=jax.ShapeDtypeStruct((2 * _NREL * _N, _D), jnp.float32),
        mesh=mesh,
        scratch_types=[
            pltpu.VMEM_SHARED((_N, _D), jnp.float32),
            pltpu.VMEM((_C,), jnp.int32),
            pltpu.VMEM((_C,), jnp.int32),
            pltpu.VMEM((_C,), jnp.int32),
            pltpu.VMEM((_C,), jnp.int32),
            pltpu.VMEM((_C, _D), jnp.float32),
            pltpu.VMEM((_C, _D), jnp.float32),
            pltpu.SemaphoreType.DMA,
            pltpu.SemaphoreType.DMA,
            pltpu.SemaphoreType.DMA,
            pltpu.SemaphoreType.DMA,
        ],
    )(xbar_flat, srcs, dsts, zeros_hbm)


def _fuse_body(p_ref, rv_ref, bias_ref, h_ref, attn_ref):
    m0 = p_ref[0] + p_ref[3]
    m1 = p_ref[1] + p_ref[4]
    m2 = p_ref[2] + p_ref[5]
    rv = rv_ref[...]
    l0 = jnp.sum(m0 * rv[0:1, :], axis=1, keepdims=True) + bias_ref[0]
    l1 = jnp.sum(m1 * rv[1:2, :], axis=1, keepdims=True) + bias_ref[1]
    l2 = jnp.sum(m2 * rv[2:3, :], axis=1, keepdims=True) + bias_ref[2]
    mx = jnp.maximum(jnp.maximum(l0, l1), l2)
    e0 = jnp.exp(l0 - mx)
    e1 = jnp.exp(l1 - mx)
    e2 = jnp.exp(l2 - mx)
    inv = 1.0 / (e0 + e1 + e2)
    a0 = e0 * inv
    a1 = e1 * inv
    a2 = e2 * inv
    h_ref[...] = jnp.maximum(a0 * m0 + a1 * m1 + a2 * m2, 0.0)
    col = lax.broadcasted_iota(jnp.int32, (1, _D), 1)
    attn_ref[0] = jnp.where(
        col == 0, jnp.sum(a0),
        jnp.where(col == 1, jnp.sum(a1),
                  jnp.where(col == 2, jnp.sum(a2), 0.0)))


def _fuse_call(partials, rv_pad, relation_bias):
    grid = _N // _BN
    return pl.pallas_call(
        _fuse_body,
        grid=(grid,),
        in_specs=[
            pl.BlockSpec((2 * _NREL, _BN, _D), lambda i: (0, i, 0)),
            pl.BlockSpec((8, _D), lambda i: (0, 0)),
            pl.BlockSpec(memory_space=pltpu.SMEM),
        ],
        out_specs=[
            pl.BlockSpec((_BN, _D), lambda i: (i, 0)),
            pl.BlockSpec((1, 1, _D), lambda i: (i, 0, 0)),
        ],
        out_shape=[
            jax.ShapeDtypeStruct((_N, _D), jnp.float32),
            jax.ShapeDtypeStruct((grid, 1, _D), jnp.float32),
        ],
    )(partials, rv_pad, relation_bias)


def kernel(h, edge_index_r0, edge_index_r1, edge_index_r2,
           relation_bias, W, node_query, rel_vector):
    q_pad = jnp.zeros((_NREL, 8, _D), jnp.float32).at[:, :_NHEADS].set(node_query)
    rv_pad = jnp.zeros((8, _D), jnp.float32).at[:_NREL].set(rel_vector)

    n_pad = _NCHUNK - _NREAL
    src_all = jnp.stack([edge_index_r0[1], edge_index_r1[1], edge_index_r2[1]])
    src_all = src_all + (jnp.arange(_NREL, dtype=jnp.int32) * _N)[:, None]
    src_pad = _ZROW + jnp.arange(n_pad * _C, dtype=jnp.int32) % 1024
    src_pad = jnp.broadcast_to(src_pad.reshape(1, n_pad, _C),
                               (_NREL, n_pad, _C))
    srcs = jnp.concatenate(
        [src_all.reshape(_NREL, _NREAL, _C), src_pad], axis=1)
    srcs = srcs.reshape(_NREL * _NCHUNK, _C)
    dsts = jnp.stack([edge_index_r0[0], edge_index_r1[0], edge_index_r2[0]])
    dst_pad = jnp.arange(n_pad * _C, dtype=jnp.int32) * 61 % _N
    dst_pad = jnp.broadcast_to(dst_pad.reshape(1, n_pad, _C),
                               (_NREL, n_pad, _C))
    dsts = jnp.concatenate(
        [dsts.reshape(_NREL, _NREAL, _C), dst_pad], axis=1)
    dsts = dsts.reshape(_NREL * _NCHUNK, _C)
    zeros_hbm = jnp.zeros((_RPT, _D), jnp.float32)

    xbar = _dense_call(h, W, q_pad)
    xbar_flat = jnp.concatenate(
        [xbar.reshape(_NREL * _N, _D), jnp.zeros((1024, _D), jnp.float32)])
    part = _sc_call(xbar_flat, srcs, dsts, zeros_hbm)
    h_next, attn_p = _fuse_call(part.reshape(2 * _NREL, _N, _D),
                                rv_pad, relation_bias)
    rel_attn_global = attn_p[:, 0, :_NREL].sum(axis=0) * (1.0 / _N)
    return (h_next, rel_attn_global)


# speedup vs baseline: 3.8107x; 1.0378x over previous
import functools

import jax
import jax.numpy as jnp
from jax import lax
from jax.experimental import pallas as pl
from jax.experimental.pallas import tpu as pltpu
from jax.experimental.pallas import tpu_sc as plsc

_N = 10000
_E = 320000
_D = 128
_NREL = 3
_NHEADS = 4

_C = 128
_NW = 32
_CPT = 80
_HALF = _CPT // 2
_NCHUNK = _NW * _CPT
_NREAL = _E // _C
_NB = 2
_NG = _HALF // _NB
_RPT = 632
_RPT_LAST = _N - 15 * _RPT
_ZROW = _NREL * _N

_BN = 1000


def _dense_body(h_ref, w_ref, q_ref, out_ref):
    hb = h_ref[...]
    for r in range(_NREL):
        acc = None
        qr = q_ref[r]
        for hd in range(_NHEADS):
            wm = w_ref[r, hd]
            hp = lax.dot_general(hb, wm, (((1,), (1,)), ((), ())),
                                 preferred_element_type=jnp.float32)
            qv = qr[hd:hd + 1, :]
            g = jnp.sum(hp * qv, axis=1, keepdims=True)
            gate = jax.nn.sigmoid(g)
            term = hp * gate
            acc = term if acc is None else acc + term
        out_ref[r] = acc * (1.0 / _NHEADS)


def _dense_call(h, W, q_pad):
    grid = _N // _BN
    return pl.pallas_call(
        _dense_body,
        grid=(grid,),
        in_specs=[
            pl.BlockSpec((_BN, _D), lambda i: (i, 0)),
            pl.BlockSpec((_NREL, _NHEADS, _D, _D), lambda i: (0, 0, 0, 0)),
            pl.BlockSpec((_NREL, 8, _D), lambda i: (0, 0, 0)),
        ],
        out_specs=pl.BlockSpec((_NREL, _BN, _D), lambda i: (0, i, 0)),
        out_shape=jax.ShapeDtypeStruct((_NREL, _N, _D), jnp.float32),
    )(h, W, q_pad)


def _sc_body(xbar_hbm, srcs_hbm, dsts_hbm, zeros_hbm, out_hbm,
             acc, si0, si1, di0, di1, rb0, rb1, mi0, mi1, mg0, mg1):
    c = lax.axis_index("c")
    s = lax.axis_index("s")
    wid = s * 2 + c
    base = s * _RPT
    sidx = (si0, si1)
    didx = (di0, di1)
    rows = (rb0, rb1)
    sem_i = (mi0, mi1)
    sem_g = (mg0, mg1)

    def idx_start(cid, k):
        pltpu.async_copy(srcs_hbm.at[cid], sidx[k], sem_i[k])
        pltpu.async_copy(dsts_hbm.at[cid], didx[k], sem_i[k])

    def idx_wait(k):
        pltpu.make_async_copy(srcs_hbm.at[0], sidx[k], sem_i[k]).wait()
        pltpu.make_async_copy(srcs_hbm.at[0], didx[k], sem_i[k]).wait()

    def gather_start(k):
        pltpu.async_copy(xbar_hbm.at[sidx[k]], rows[k], sem_g[k])

    def gather_wait(k):
        pltpu.make_async_copy(xbar_hbm.at[pl.ds(0, _C)], rows[k],
                              sem_g[k]).wait()

    for r in range(_NREL):
        @pl.when(s < 15)
        def _():
            pltpu.sync_copy(zeros_hbm, acc.at[pl.ds(base, _RPT)])

        @pl.when(s == 15)
        def _():
            pltpu.sync_copy(zeros_hbm.at[pl.ds(0, _RPT_LAST)],
                            acc.at[pl.ds(base, _RPT_LAST)])

        plsc.subcore_barrier()

        tbase = r * _NCHUNK + wid * _CPT
        idx_start(tbase, 0)
        idx_start(tbase + 1, 1)
        idx_wait(0)
        gather_start(0)

        def outer(g, carry):
            for b in range(_NB):
                j = g * _NB + b

                @pl.when(j < _CPT - 1)
                def _():
                    idx_wait(1 - b)
                    gather_start(1 - b)

                gather_wait(b)
                pltpu.sync_copy(rows[b], acc.at[didx[b]], add=True)

                @pl.when(j < _CPT - 2)
                def _():
                    idx_start(tbase + j + 2, b)
            return carry

        lax.fori_loop(0, _CPT // _NB, outer, 0)
        plsc.subcore_barrier()

        out_off = (c * _NREL + r) * _N + base

        @pl.when(s < 15)
        def _():
            pltpu.sync_copy(acc.at[pl.ds(base, _RPT)],
                            out_hbm.at[pl.ds(out_off, _RPT)])

        @pl.when(s == 15)
        def _():
            pltpu.sync_copy(acc.at[pl.ds(base, _RPT_LAST)],
                            out_hbm.at[pl.ds(out_off, _RPT_LAST)])

        plsc.subcore_barrier()


def _sc_call(xbar_flat, srcs, dsts, zeros_hbm):
    mesh = plsc.VectorSubcoreMesh(core_axis_name="c", subcore_axis_name="s")
    return pl.kernel(
        _sc_body,
        out_type=jax.ShapeDtypeStruct((2 * _NREL * _N, _D), jnp.float32),
        mesh=mesh,
        scratch_types=[
            pltpu.VMEM_SHARED((_N, _D), jnp.float32),
            pltpu.VMEM((_C,), jnp.int32),
            pltpu.VMEM((_C,), jnp.int32),
            pltpu.VMEM((_C,), jnp.int32),
            pltpu.VMEM((_C,), jnp.int32),
            pltpu.VMEM((_C, _D), jnp.float32),
            pltpu.VMEM((_C, _D), jnp.float32),
            pltpu.SemaphoreType.DMA,
            pltpu.SemaphoreType.DMA,
            pltpu.SemaphoreType.DMA,
            pltpu.SemaphoreType.DMA,
        ],
    )(xbar_flat, srcs, dsts, zeros_hbm)


def _fuse_body(p_ref, rv_ref, bias_ref, h_ref, attn_ref):
    m0 = p_ref[0] + p_ref[3]
    m1 = p_ref[1] + p_ref[4]
    m2 = p_ref[2] + p_ref[5]
    rv = rv_ref[...]
    l0 = jnp.sum(m0 * rv[0:1, :], axis=1, keepdims=True) + bias_ref[0]
    l1 = jnp.sum(m1 * rv[1:2, :], axis=1, keepdims=True) + bias_ref[1]
    l2 = jnp.sum(m2 * rv[2:3, :], axis=1, keepdims=True) + bias_ref[2]
    mx = jnp.maximum(jnp.maximum(l0, l1), l2)
    e0 = jnp.exp(l0 - mx)
    e1 = jnp.exp(l1 - mx)
    e2 = jnp.exp(l2 - mx)
    inv = 1.0 / (e0 + e1 + e2)
    a0 = e0 * inv
    a1 = e1 * inv
    a2 = e2 * inv
    h_ref[...] = jnp.maximum(a0 * m0 + a1 * m1 + a2 * m2, 0.0)
    col = lax.broadcasted_iota(jnp.int32, (1, _D), 1)
    attn_ref[0] = jnp.where(
        col == 0, jnp.sum(a0),
        jnp.where(col == 1, jnp.sum(a1),
                  jnp.where(col == 2, jnp.sum(a2), 0.0)))


def _fuse_call(partials, rv_pad, relation_bias):
    grid = _N // _BN
    return pl.pallas_call(
        _fuse_body,
        grid=(grid,),
        in_specs=[
            pl.BlockSpec((2 * _NREL, _BN, _D), lambda i: (0, i, 0)),
            pl.BlockSpec((8, _D), lambda i: (0, 0)),
            pl.BlockSpec(memory_space=pltpu.SMEM),
        ],
        out_specs=[
            pl.BlockSpec((_BN, _D), lambda i: (i, 0)),
            pl.BlockSpec((1, 1, _D), lambda i: (i, 0, 0)),
        ],
        out_shape=[
            jax.ShapeDtypeStruct((_N, _D), jnp.float32),
            jax.ShapeDtypeStruct((grid, 1, _D), jnp.float32),
        ],
    )(partials, rv_pad, relation_bias)


def kernel(h, edge_index_r0, edge_index_r1, edge_index_r2,
           relation_bias, W, node_query, rel_vector):
    q_pad = jnp.zeros((_NREL, 8, _D), jnp.float32).at[:, :_NHEADS].set(node_query)
    rv_pad = jnp.zeros((8, _D), jnp.float32).at[:_NREL].set(rel_vector)

    n_pad = _NCHUNK - _NREAL
    src_all = jnp.stack([edge_index_r0[1], edge_index_r1[1], edge_index_r2[1]])
    src_all = src_all + (jnp.arange(_NREL, dtype=jnp.int32) * _N)[:, None]
    src_pad = _ZROW + jnp.arange(n_pad * _C, dtype=jnp.int32) % 1024
    src_pad = jnp.broadcast_to(src_pad.reshape(1, n_pad, _C),
                               (_NREL, n_pad, _C))
    srcs = jnp.concatenate(
        [src_all.reshape(_NREL, _NREAL, _C), src_pad], axis=1)
    srcs = srcs.reshape(_NREL * _NCHUNK, _C)
    dsts = jnp.stack([edge_index_r0[0], edge_index_r1[0], edge_index_r2[0]])
    dst_pad = jnp.arange(n_pad * _C, dtype=jnp.int32) * 61 % _N
    dst_pad = jnp.broadcast_to(dst_pad.reshape(1, n_pad, _C),
                               (_NREL, n_pad, _C))
    dsts = jnp.concatenate(
        [dsts.reshape(_NREL, _NREAL, _C), dst_pad], axis=1)
    dsts = dsts.reshape(_NREL * _NCHUNK, _C)
    zeros_hbm = jnp.zeros((_RPT, _D), jnp.float32)

    xbar = _dense_call(h, W, q_pad)
    xbar_flat = jnp.concatenate(
        [xbar.reshape(_NREL * _N, _D), jnp.zeros((1024, _D), jnp.float32)])
    part = _sc_call(xbar_flat, srcs, dsts, zeros_hbm)
    h_next, attn_p = _fuse_call(part.reshape(2 * _NREL, _N, _D),
                                rv_pad, relation_bias)
    rel_attn_global = attn_p[:, 0, :_NREL].sum(axis=0) * (1.0 / _N)
    return (h_next, rel_attn_global)


# speedup vs baseline: 4.2334x; 1.1109x over previous
import functools

import jax
import jax.numpy as jnp
from jax import lax
from jax.experimental import pallas as pl
from jax.experimental.pallas import tpu as pltpu
from jax.experimental.pallas import tpu_sc as plsc

_N = 10000
_E = 320000
_D = 128
_NREL = 3
_NHEADS = 4

_C = 128
_NW = 32
_CPT = 80
_HALF = _CPT // 2
_NCHUNK = _NW * _CPT
_NREAL = _E // _C
_NB = 2
_NG = _HALF // _NB
_RPT = 632
_RPT_LAST = _N - 15 * _RPT
_ZROW = _NREL * _N

_BN = 1000


def _dense_body(h_ref, w_ref, q_ref, out_ref):
    hb = h_ref[...]
    for r in range(_NREL):
        acc = None
        qr = q_ref[r]
        for hd in range(_NHEADS):
            wm = w_ref[r, hd]
            hp = lax.dot_general(hb, wm, (((1,), (1,)), ((), ())),
                                 preferred_element_type=jnp.float32)
            qv = qr[hd:hd + 1, :]
            g = jnp.sum(hp * qv, axis=1, keepdims=True)
            gate = jax.nn.sigmoid(g)
            term = hp * gate
            acc = term if acc is None else acc + term
        out_ref[r] = acc * (1.0 / _NHEADS)


def _dense_call(h, W, q_pad):
    grid = _N // _BN
    return pl.pallas_call(
        _dense_body,
        grid=(grid,),
        in_specs=[
            pl.BlockSpec((_BN, _D), lambda i: (i, 0)),
            pl.BlockSpec((_NREL, _NHEADS, _D, _D), lambda i: (0, 0, 0, 0)),
            pl.BlockSpec((_NREL, 8, _D), lambda i: (0, 0, 0)),
        ],
        out_specs=pl.BlockSpec((_NREL, _BN, _D), lambda i: (0, i, 0)),
        out_shape=jax.ShapeDtypeStruct((_NREL, _N, _D), jnp.float32),
    )(h, W, q_pad)


def _sc_body(xbar_hbm, srcs_hbm, dsts_hbm, zeros_hbm, out_hbm, acc,
             si0, si1, si2, si3, di0, di1, di2, di3, rb0, rb1,
             mi0, mi1, mi2, mi3, mg0, mg1, ms0, ms1):
    c = lax.axis_index("c")
    s = lax.axis_index("s")
    wid = s * 2 + c
    base = s * _RPT
    sidx = (si0, si1, si2, si3)
    didx = (di0, di1, di2, di3)
    rows = (rb0, rb1)
    sem_i = (mi0, mi1, mi2, mi3)
    sem_g = (mg0, mg1)
    sem_s = (ms0, ms1)

    def idx_start(cid, k):
        pltpu.async_copy(srcs_hbm.at[cid], sidx[k], sem_i[k])
        pltpu.async_copy(dsts_hbm.at[cid], didx[k], sem_i[k])

    def idx_wait(k):
        pltpu.make_async_copy(srcs_hbm.at[0], sidx[k], sem_i[k]).wait()
        pltpu.make_async_copy(srcs_hbm.at[0], didx[k], sem_i[k]).wait()

    def gather_start(k4, r2):
        pltpu.async_copy(xbar_hbm.at[sidx[k4]], rows[r2], sem_g[r2])

    def gather_wait(r2):
        pltpu.make_async_copy(xbar_hbm.at[pl.ds(0, _C)], rows[r2],
                              sem_g[r2]).wait()

    def scatter_start(r2, k4):
        pltpu.async_copy(rows[r2], acc.at[didx[k4]], sem_s[r2], add=True)

    def scatter_wait(r2):
        pltpu.make_async_copy(xbar_hbm.at[pl.ds(0, _C)], rows[r2],
                              sem_s[r2]).wait()

    for r in range(_NREL):
        @pl.when(s < 15)
        def _():
            pltpu.sync_copy(zeros_hbm, acc.at[pl.ds(base, _RPT)])

        @pl.when(s == 15)
        def _():
            pltpu.sync_copy(zeros_hbm.at[pl.ds(0, _RPT_LAST)],
                            acc.at[pl.ds(base, _RPT_LAST)])

        plsc.subcore_barrier()

        tbase = r * _NCHUNK + wid * _CPT
        idx_start(tbase, 0)
        idx_start(tbase + 1, 1)
        idx_start(tbase + 2, 2)
        idx_wait(0)
        gather_start(0, 0)

        def outer(g, carry):
            for b4 in range(4):
                j = g * 4 + b4
                r2 = b4 % 2

                @pl.when(j > 0)
                def _():
                    scatter_wait(1 - r2)

                @pl.when(j < _CPT - 1)
                def _():
                    idx_wait((b4 + 1) % 4)
                    gather_start((b4 + 1) % 4, 1 - r2)

                @pl.when(j < _CPT - 3)
                def _():
                    idx_start(tbase + j + 3, (b4 + 3) % 4)

                gather_wait(r2)
                scatter_start(r2, b4)
            return carry

        lax.fori_loop(0, _CPT // 4, outer, 0)
        scatter_wait(1)
        plsc.subcore_barrier()

        out_off = (c * _NREL + r) * _N + base

        @pl.when(s < 15)
        def _():
            pltpu.sync_copy(acc.at[pl.ds(base, _RPT)],
                            out_hbm.at[pl.ds(out_off, _RPT)])

        @pl.when(s == 15)
        def _():
            pltpu.sync_copy(acc.at[pl.ds(base, _RPT_LAST)],
                            out_hbm.at[pl.ds(out_off, _RPT_LAST)])

        plsc.subcore_barrier()


def _sc_call(xbar_flat, srcs, dsts, zeros_hbm):
    mesh = plsc.VectorSubcoreMesh(core_axis_name="c", subcore_axis_name="s")
    return pl.kernel(
        _sc_body,
        out_type=jax.ShapeDtypeStruct((2 * _NREL * _N, _D), jnp.float32),
        mesh=mesh,
        scratch_types=[
            pltpu.VMEM_SHARED((_N, _D), jnp.float32),
            pltpu.VMEM((_C,), jnp.int32),
            pltpu.VMEM((_C,), jnp.int32),
            pltpu.VMEM((_C,), jnp.int32),
            pltpu.VMEM((_C,), jnp.int32),
            pltpu.VMEM((_C,), jnp.int32),
            pltpu.VMEM((_C,), jnp.int32),
            pltpu.VMEM((_C,), jnp.int32),
            pltpu.VMEM((_C,), jnp.int32),
            pltpu.VMEM((_C, _D), jnp.float32),
            pltpu.VMEM((_C, _D), jnp.float32),
            pltpu.SemaphoreType.DMA,
            pltpu.SemaphoreType.DMA,
            pltpu.SemaphoreType.DMA,
            pltpu.SemaphoreType.DMA,
            pltpu.SemaphoreType.DMA,
            pltpu.SemaphoreType.DMA,
            pltpu.SemaphoreType.DMA,
            pltpu.SemaphoreType.DMA,
        ],
    )(xbar_flat, srcs, dsts, zeros_hbm)


def _fuse_body(p_ref, rv_ref, bias_ref, h_ref, attn_ref):
    m0 = p_ref[0] + p_ref[3]
    m1 = p_ref[1] + p_ref[4]
    m2 = p_ref[2] + p_ref[5]
    rv = rv_ref[...]
    l0 = jnp.sum(m0 * rv[0:1, :], axis=1, keepdims=True) + bias_ref[0]
    l1 = jnp.sum(m1 * rv[1:2, :], axis=1, keepdims=True) + bias_ref[1]
    l2 = jnp.sum(m2 * rv[2:3, :], axis=1, keepdims=True) + bias_ref[2]
    mx = jnp.maximum(jnp.maximum(l0, l1), l2)
    e0 = jnp.exp(l0 - mx)
    e1 = jnp.exp(l1 - mx)
    e2 = jnp.exp(l2 - mx)
    inv = 1.0 / (e0 + e1 + e2)
    a0 = e0 * inv
    a1 = e1 * inv
    a2 = e2 * inv
    h_ref[...] = jnp.maximum(a0 * m0 + a1 * m1 + a2 * m2, 0.0)
    col = lax.broadcasted_iota(jnp.int32, (1, _D), 1)
    attn_ref[0] = jnp.where(
        col == 0, jnp.sum(a0),
        jnp.where(col == 1, jnp.sum(a1),
                  jnp.where(col == 2, jnp.sum(a2), 0.0)))


def _fuse_call(partials, rv_pad, relation_bias):
    grid = _N // _BN
    return pl.pallas_call(
        _fuse_body,
        grid=(grid,),
        in_specs=[
            pl.BlockSpec((2 * _NREL, _BN, _D), lambda i: (0, i, 0)),
            pl.BlockSpec((8, _D), lambda i: (0, 0)),
            pl.BlockSpec(memory_space=pltpu.SMEM),
        ],
        out_specs=[
            pl.BlockSpec((_BN, _D), lambda i: (i, 0)),
            pl.BlockSpec((1, 1, _D), lambda i: (i, 0, 0)),
        ],
        out_shape=[
            jax.ShapeDtypeStruct((_N, _D), jnp.float32),
            jax.ShapeDtypeStruct((grid, 1, _D), jnp.float32),
        ],
    )(partials, rv_pad, relation_bias)


def kernel(h, edge_index_r0, edge_index_r1, edge_index_r2,
           relation_bias, W, node_query, rel_vector):
    q_pad = jnp.zeros((_NREL, 8, _D), jnp.float32).at[:, :_NHEADS].set(node_query)
    rv_pad = jnp.zeros((8, _D), jnp.float32).at[:_NREL].set(rel_vector)

    n_pad = _NCHUNK - _NREAL
    src_all = jnp.stack([edge_index_r0[1], edge_index_r1[1], edge_index_r2[1]])
    src_all = src_all + (jnp.arange(_NREL, dtype=jnp.int32) * _N)[:, None]
    src_pad = _ZROW + jnp.arange(n_pad * _C, dtype=jnp.int32) % 1024
    src_pad = jnp.broadcast_to(src_pad.reshape(1, n_pad, _C),
                               (_NREL, n_pad, _C))
    srcs = jnp.concatenate(
        [src_all.reshape(_NREL, _NREAL, _C), src_pad], axis=1)
    srcs = srcs.reshape(_NREL * _NCHUNK, _C)
    dsts = jnp.stack([edge_index_r0[0], edge_index_r1[0], edge_index_r2[0]])
    dst_pad = jnp.arange(n_pad * _C, dtype=jnp.int32) * 61 % _N
    dst_pad = jnp.broadcast_to(dst_pad.reshape(1, n_pad, _C),
                               (_NREL, n_pad, _C))
    dsts = jnp.concatenate(
        [dsts.reshape(_NREL, _NREAL, _C), dst_pad], axis=1)
    dsts = dsts.reshape(_NREL * _NCHUNK, _C)
    zeros_hbm = jnp.zeros((_RPT, _D), jnp.float32)

    xbar = _dense_call(h, W, q_pad)
    xbar_flat = jnp.concatenate(
        [xbar.reshape(_NREL * _N, _D), jnp.zeros((1024, _D), jnp.float32)])
    part = _sc_call(xbar_flat, srcs, dsts, zeros_hbm)
    h_next, attn_p = _fuse_call(part.reshape(2 * _NREL, _N, _D),
                                rv_pad, relation_bias)
    rel_attn_global = attn_p[:, 0, :_NREL].sum(axis=0) * (1.0 / _N)
    return (h_next, rel_attn_global)


# speedup vs baseline: 4.3290x; 1.0226x over previous
import functools

import jax
import jax.numpy as jnp
from jax import lax
from jax.experimental import pallas as pl
from jax.experimental.pallas import tpu as pltpu
from jax.experimental.pallas import tpu_sc as plsc

_N = 10000
_E = 320000
_D = 128
_NREL = 3
_NHEADS = 4

_C = 128
_NW = 32
_CPT = 80
_HALF = _CPT // 2
_NCHUNK = _NW * _CPT
_NREAL = _E // _C
_NB = 2
_NG = _HALF // _NB
_RPT = 632
_RPT_LAST = _N - 15 * _RPT
_APAD = 240

_BN = 1000


def _dense_body(h_ref, w_ref, q_ref, out_ref):
    hb = h_ref[...]
    for r in range(_NREL):
        acc = None
        qr = q_ref[r]
        for hd in range(_NHEADS):
            wm = w_ref[r, hd]
            hp = lax.dot_general(hb, wm, (((1,), (1,)), ((), ())),
                                 preferred_element_type=jnp.float32)
            qv = qr[hd:hd + 1, :]
            g = jnp.sum(hp * qv, axis=1, keepdims=True)
            gate = jax.nn.sigmoid(g)
            term = hp * gate
            acc = term if acc is None else acc + term
        out_ref[r] = acc * (1.0 / _NHEADS)


def _dense_call(h, W, q_pad):
    grid = _N // _BN
    return pl.pallas_call(
        _dense_body,
        grid=(grid,),
        in_specs=[
            pl.BlockSpec((_BN, _D), lambda i: (i, 0)),
            pl.BlockSpec((_NREL, _NHEADS, _D, _D), lambda i: (0, 0, 0, 0)),
            pl.BlockSpec((_NREL, 8, _D), lambda i: (0, 0, 0)),
        ],
        out_specs=pl.BlockSpec((_NREL, _BN, _D), lambda i: (0, i, 0)),
        out_shape=jax.ShapeDtypeStruct((_NREL, _N, _D), jnp.float32),
    )(h, W, q_pad)


def _sc_body(xbar_hbm, srcs_hbm, dsts_hbm, zeros_hbm, out_hbm, acc,
             si0, si1, si2, si3, di0, di1, di2, di3, rb0, rb1,
             mi0, mi1, mi2, mi3, mg0, mg1, ms0, ms1):
    c = lax.axis_index("c")
    s = lax.axis_index("s")
    wid = s * 2 + c
    base = s * _RPT
    sidx = (si0, si1, si2, si3)
    didx = (di0, di1, di2, di3)
    rows = (rb0, rb1)
    sem_i = (mi0, mi1, mi2, mi3)
    sem_g = (mg0, mg1)
    sem_s = (ms0, ms1)

    def idx_start(cid, k):
        pltpu.async_copy(srcs_hbm.at[cid], sidx[k], sem_i[k])
        pltpu.async_copy(dsts_hbm.at[cid], didx[k], sem_i[k])

    def idx_wait(k):
        pltpu.make_async_copy(srcs_hbm.at[0], sidx[k], sem_i[k]).wait()
        pltpu.make_async_copy(srcs_hbm.at[0], didx[k], sem_i[k]).wait()

    def gather_start(k4, r2):
        pltpu.async_copy(xbar_hbm.at[sidx[k4]], rows[r2], sem_g[r2])

    def gather_wait(r2):
        pltpu.make_async_copy(xbar_hbm.at[pl.ds(0, _C)], rows[r2],
                              sem_g[r2]).wait()

    def scatter_start(r2, k4):
        pltpu.async_copy(rows[r2], acc.at[didx[k4]], sem_s[r2], add=True)

    def scatter_wait(r2):
        pltpu.make_async_copy(xbar_hbm.at[pl.ds(0, _C)], rows[r2],
                              sem_s[r2]).wait()

    for r in range(_NREL):
        @pl.when(s < 15)
        def _():
            pltpu.sync_copy(zeros_hbm, acc.at[pl.ds(base, _RPT)])

        @pl.when(s == 15)
        def _():
            pltpu.sync_copy(zeros_hbm.at[pl.ds(0, _RPT_LAST)],
                            acc.at[pl.ds(base, _RPT_LAST)])

        plsc.subcore_barrier()

        tbase = r * _NCHUNK + wid * _CPT
        idx_start(tbase, 0)
        idx_start(tbase + 1, 1)
        idx_start(tbase + 2, 2)
        idx_wait(0)
        gather_start(0, 0)

        def outer(g, carry):
            for b4 in range(4):
                j = g * 4 + b4
                r2 = b4 % 2

                @pl.when(j > 0)
                def _():
                    scatter_wait(1 - r2)

                @pl.when(j < _CPT - 1)
                def _():
                    idx_wait((b4 + 1) % 4)
                    gather_start((b4 + 1) % 4, 1 - r2)

                @pl.when(j < _CPT - 3)
                def _():
                    idx_start(tbase + j + 3, (b4 + 3) % 4)

                gather_wait(r2)
                scatter_start(r2, b4)
            return carry

        lax.fori_loop(0, _CPT // 4, outer, 0)
        scatter_wait(1)
        plsc.subcore_barrier()

        out_off = (c * _NREL + r) * _N + base

        @pl.when(s < 15)
        def _():
            pltpu.sync_copy(acc.at[pl.ds(base, _RPT)],
                            out_hbm.at[pl.ds(out_off, _RPT)])

        @pl.when(s == 15)
        def _():
            pltpu.sync_copy(acc.at[pl.ds(base, _RPT_LAST)],
                            out_hbm.at[pl.ds(out_off, _RPT_LAST)])

        plsc.subcore_barrier()


def _sc_call(xbar_flat, srcs, dsts, zeros_hbm):
    mesh = plsc.VectorSubcoreMesh(core_axis_name="c", subcore_axis_name="s")
    return pl.kernel(
        _sc_body,
        out_type=jax.ShapeDtypeStruct((2 * _NREL * _N, _D), jnp.float32),
        mesh=mesh,
        scratch_types=[
            pltpu.VMEM_SHARED((_N + _APAD, _D), jnp.float32),
            pltpu.VMEM((_C,), jnp.int32),
            pltpu.VMEM((_C,), jnp.int32),
            pltpu.VMEM((_C,), jnp.int32),
            pltpu.VMEM((_C,), jnp.int32),
            pltpu.VMEM((_C,), jnp.int32),
            pltpu.VMEM((_C,), jnp.int32),
            pltpu.VMEM((_C,), jnp.int32),
            pltpu.VMEM((_C,), jnp.int32),
            pltpu.VMEM((_C, _D), jnp.float32),
            pltpu.VMEM((_C, _D), jnp.float32),
            pltpu.SemaphoreType.DMA,
            pltpu.SemaphoreType.DMA,
            pltpu.SemaphoreType.DMA,
            pltpu.SemaphoreType.DMA,
            pltpu.SemaphoreType.DMA,
            pltpu.SemaphoreType.DMA,
            pltpu.SemaphoreType.DMA,
            pltpu.SemaphoreType.DMA,
        ],
    )(xbar_flat, srcs, dsts, zeros_hbm)


def _fuse_body(p_ref, rv_ref, bias_ref, h_ref, attn_ref):
    m0 = p_ref[0] + p_ref[3]
    m1 = p_ref[1] + p_ref[4]
    m2 = p_ref[2] + p_ref[5]
    rv = rv_ref[...]
    l0 = jnp.sum(m0 * rv[0:1, :], axis=1, keepdims=True) + bias_ref[0]
    l1 = jnp.sum(m1 * rv[1:2, :], axis=1, keepdims=True) + bias_ref[1]
    l2 = jnp.sum(m2 * rv[2:3, :], axis=1, keepdims=True) + bias_ref[2]
    mx = jnp.maximum(jnp.maximum(l0, l1), l2)
    e0 = jnp.exp(l0 - mx)
    e1 = jnp.exp(l1 - mx)
    e2 = jnp.exp(l2 - mx)
    inv = 1.0 / (e0 + e1 + e2)
    a0 = e0 * inv
    a1 = e1 * inv
    a2 = e2 * inv
    h_ref[...] = jnp.maximum(a0 * m0 + a1 * m1 + a2 * m2, 0.0)
    col = lax.broadcasted_iota(jnp.int32, (1, _D), 1)
    attn_ref[0] = jnp.where(
        col == 0, jnp.sum(a0),
        jnp.where(col == 1, jnp.sum(a1),
                  jnp.where(col == 2, jnp.sum(a2), 0.0)))


def _fuse_call(partials, rv_pad, relation_bias):
    grid = _N // _BN
    return pl.pallas_call(
        _fuse_body,
        grid=(grid,),
        in_specs=[
            pl.BlockSpec((2 * _NREL, _BN, _D), lambda i: (0, i, 0)),
            pl.BlockSpec((8, _D), lambda i: (0, 0)),
            pl.BlockSpec(memory_space=pltpu.SMEM),
        ],
        out_specs=[
            pl.BlockSpec((_BN, _D), lambda i: (i, 0)),
            pl.BlockSpec((1, 1, _D), lambda i: (i, 0, 0)),
        ],
        out_shape=[
            jax.ShapeDtypeStruct((_N, _D), jnp.float32),
            jax.ShapeDtypeStruct((grid, 1, _D), jnp.float32),
        ],
    )(partials, rv_pad, relation_bias)


def kernel(h, edge_index_r0, edge_index_r1, edge_index_r2,
           relation_bias, W, node_query, rel_vector):
    q_pad = jnp.zeros((_NREL, 8, _D), jnp.float32).at[:, :_NHEADS].set(node_query)
    rv_pad = jnp.zeros((8, _D), jnp.float32).at[:_NREL].set(rel_vector)

    n_pad = _NCHUNK - _NREAL
    src_all = jnp.stack([edge_index_r0[1], edge_index_r1[1], edge_index_r2[1]])
    src_all = src_all + (jnp.arange(_NREL, dtype=jnp.int32) * _N)[:, None]
    src_pad = jnp.arange(n_pad * _C, dtype=jnp.int32) * 37 % _N
    src_pad = jnp.broadcast_to(src_pad.reshape(1, n_pad, _C),
                               (_NREL, n_pad, _C))
    srcs = jnp.concatenate(
        [src_all.reshape(_NREL, _NREAL, _C), src_pad], axis=1)
    srcs = srcs.reshape(_NREL * _NCHUNK, _C)
    dsts = jnp.stack([edge_index_r0[0], edge_index_r1[0], edge_index_r2[0]])
    dst_pad = _N + jnp.arange(n_pad * _C, dtype=jnp.int32) % _APAD
    dst_pad = jnp.broadcast_to(dst_pad.reshape(1, n_pad, _C),
                               (_NREL, n_pad, _C))
    dsts = jnp.concatenate(
        [dsts.reshape(_NREL, _NREAL, _C), dst_pad], axis=1)
    dsts = dsts.reshape(_NREL * _NCHUNK, _C)
    zeros_hbm = jnp.zeros((_RPT, _D), jnp.float32)

    xbar = _dense_call(h, W, q_pad)
    part = _sc_call(xbar.reshape(_NREL * _N, _D), srcs, dsts, zeros_hbm)
    h_next, attn_p = _fuse_call(part.reshape(2 * _NREL, _N, _D),
                                rv_pad, relation_bias)
    rel_attn_global = attn_p[:, 0, :_NREL].sum(axis=0) * (1.0 / _N)
    return (h_next, rel_attn_global)


# speedup vs baseline: 4.3362x; 1.0016x over previous
import functools

import jax
import jax.numpy as jnp
from jax import lax
from jax.experimental import pallas as pl
from jax.experimental.pallas import tpu as pltpu
from jax.experimental.pallas import tpu_sc as plsc

_N = 10000
_E = 320000
_D = 128
_NREL = 3
_NHEADS = 4

_C = 128
_NW = 32
_CPT = 80
_HALF = _CPT // 2
_NCHUNK = _NW * _CPT
_NREAL = _E // _C
_NB = 2
_NG = _HALF // _NB
_RPT = 632
_RPT_LAST = _N - 15 * _RPT
_APAD = 240

_BN = 1000


def _dense_body(h_ref, w_ref, q_ref, out_ref):
    hb = h_ref[...]
    for r in range(_NREL):
        acc = None
        qr = q_ref[r]
        for hd in range(_NHEADS):
            wm = w_ref[r, hd]
            hp = lax.dot_general(hb, wm, (((1,), (1,)), ((), ())),
                                 preferred_element_type=jnp.float32)
            qv = qr[hd:hd + 1, :]
            g = jnp.sum(hp * qv, axis=1, keepdims=True)
            gate = jax.nn.sigmoid(g)
            term = hp * gate
            acc = term if acc is None else acc + term
        out_ref[r] = acc * (1.0 / _NHEADS)


def _dense_call(h, W, q_pad):
    grid = _N // _BN
    return pl.pallas_call(
        _dense_body,
        grid=(grid,),
        in_specs=[
            pl.BlockSpec((_BN, _D), lambda i: (i, 0)),
            pl.BlockSpec((_NREL, _NHEADS, _D, _D), lambda i: (0, 0, 0, 0)),
            pl.BlockSpec((_NREL, 8, _D), lambda i: (0, 0, 0)),
        ],
        out_specs=pl.BlockSpec((_NREL, _BN, _D), lambda i: (0, i, 0)),
        out_shape=jax.ShapeDtypeStruct((_NREL, _N, _D), jnp.float32),
    )(h.astype(jnp.bfloat16), W.astype(jnp.bfloat16), q_pad)


def _sc_body(xbar_hbm, srcs_hbm, dsts_hbm, zeros_hbm, out_hbm, acc,
             si0, si1, si2, si3, di0, di1, di2, di3, rb0, rb1,
             mi0, mi1, mi2, mi3, mg0, mg1, ms0, ms1):
    c = lax.axis_index("c")
    s = lax.axis_index("s")
    wid = s * 2 + c
    base = s * _RPT
    sidx = (si0, si1, si2, si3)
    didx = (di0, di1, di2, di3)
    rows = (rb0, rb1)
    sem_i = (mi0, mi1, mi2, mi3)
    sem_g = (mg0, mg1)
    sem_s = (ms0, ms1)

    def idx_start(cid, k):
        pltpu.async_copy(srcs_hbm.at[cid], sidx[k], sem_i[k])
        pltpu.async_copy(dsts_hbm.at[cid], didx[k], sem_i[k])

    def idx_wait(k):
        pltpu.make_async_copy(srcs_hbm.at[0], sidx[k], sem_i[k]).wait()
        pltpu.make_async_copy(srcs_hbm.at[0], didx[k], sem_i[k]).wait()

    def gather_start(k4, r2):
        pltpu.async_copy(xbar_hbm.at[sidx[k4]], rows[r2], sem_g[r2])

    def gather_wait(r2):
        pltpu.make_async_copy(xbar_hbm.at[pl.ds(0, _C)], rows[r2],
                              sem_g[r2]).wait()

    def scatter_start(r2, k4):
        pltpu.async_copy(rows[r2], acc.at[didx[k4]], sem_s[r2], add=True)

    def scatter_wait(r2):
        pltpu.make_async_copy(xbar_hbm.at[pl.ds(0, _C)], rows[r2],
                              sem_s[r2]).wait()

    def zero_slice():
        @pl.when(s < 15)
        def _():
            pltpu.sync_copy(zeros_hbm, acc.at[pl.ds(base, _RPT)])

        @pl.when(s == 15)
        def _():
            pltpu.sync_copy(zeros_hbm.at[pl.ds(0, _RPT_LAST)],
                            acc.at[pl.ds(base, _RPT_LAST)])

    zero_slice()
    plsc.subcore_barrier()

    for r in range(_NREL):
        tbase = r * _NCHUNK + wid * _CPT
        idx_start(tbase, 0)
        idx_start(tbase + 1, 1)
        idx_start(tbase + 2, 2)
        idx_wait(0)
        gather_start(0, 0)

        def outer(g, carry):
            for b4 in range(4):
                j = g * 4 + b4
                r2 = b4 % 2

                @pl.when(j > 0)
                def _():
                    scatter_wait(1 - r2)

                @pl.when(j < _CPT - 1)
                def _():
                    idx_wait((b4 + 1) % 4)
                    gather_start((b4 + 1) % 4, 1 - r2)

                @pl.when(j < _CPT - 3)
                def _():
                    idx_start(tbase + j + 3, (b4 + 3) % 4)

                gather_wait(r2)
                scatter_start(r2, b4)
            return carry

        lax.fori_loop(0, _CPT // 4, outer, 0)
        scatter_wait(1)
        plsc.subcore_barrier()

        out_off = (c * _NREL + r) * _N + base

        @pl.when(s < 15)
        def _():
            pltpu.sync_copy(acc.at[pl.ds(base, _RPT)],
                            out_hbm.at[pl.ds(out_off, _RPT)])

        @pl.when(s == 15)
        def _():
            pltpu.sync_copy(acc.at[pl.ds(base, _RPT_LAST)],
                            out_hbm.at[pl.ds(out_off, _RPT_LAST)])

        if r < _NREL - 1:
            zero_slice()
        plsc.subcore_barrier()


def _sc_call(xbar_flat, srcs, dsts, zeros_hbm):
    mesh = plsc.VectorSubcoreMesh(core_axis_name="c", subcore_axis_name="s")
    return pl.kernel(
        _sc_body,
        out_type=jax.ShapeDtypeStruct((2 * _NREL * _N, _D), jnp.float32),
        mesh=mesh,
        scratch_types=[
            pltpu.VMEM_SHARED((_N + _APAD, _D), jnp.float32),
            pltpu.VMEM((_C,), jnp.int32),
            pltpu.VMEM((_C,), jnp.int32),
            pltpu.VMEM((_C,), jnp.int32),
            pltpu.VMEM((_C,), jnp.int32),
            pltpu.VMEM((_C,), jnp.int32),
            pltpu.VMEM((_C,), jnp.int32),
            pltpu.VMEM((_C,), jnp.int32),
            pltpu.VMEM((_C,), jnp.int32),
            pltpu.VMEM((_C, _D), jnp.float32),
            pltpu.VMEM((_C, _D), jnp.float32),
            pltpu.SemaphoreType.DMA,
            pltpu.SemaphoreType.DMA,
            pltpu.SemaphoreType.DMA,
            pltpu.SemaphoreType.DMA,
            pltpu.SemaphoreType.DMA,
            pltpu.SemaphoreType.DMA,
            pltpu.SemaphoreType.DMA,
            pltpu.SemaphoreType.DMA,
        ],
    )(xbar_flat, srcs, dsts, zeros_hbm)


def _fuse_body(p_ref, rv_ref, bias_ref, h_ref, attn_ref):
    m0 = p_ref[0] + p_ref[3]
    m1 = p_ref[1] + p_ref[4]
    m2 = p_ref[2] + p_ref[5]
    rv = rv_ref[...]
    l0 = jnp.sum(m0 * rv[0:1, :], axis=1, keepdims=True) + bias_ref[0]
    l1 = jnp.sum(m1 * rv[1:2, :], axis=1, keepdims=True) + bias_ref[1]
    l2 = jnp.sum(m2 * rv[2:3, :], axis=1, keepdims=True) + bias_ref[2]
    mx = jnp.maximum(jnp.maximum(l0, l1), l2)
    e0 = jnp.exp(l0 - mx)
    e1 = jnp.exp(l1 - mx)
    e2 = jnp.exp(l2 - mx)
    inv = 1.0 / (e0 + e1 + e2)
    a0 = e0 * inv
    a1 = e1 * inv
    a2 = e2 * inv
    h_ref[...] = jnp.maximum(a0 * m0 + a1 * m1 + a2 * m2, 0.0)
    col = lax.broadcasted_iota(jnp.int32, (1, _D), 1)
    attn_ref[0] = jnp.where(
        col == 0, jnp.sum(a0),
        jnp.where(col == 1, jnp.sum(a1),
                  jnp.where(col == 2, jnp.sum(a2), 0.0)))


def _fuse_call(partials, rv_pad, relation_bias):
    grid = _N // _BN
    return pl.pallas_call(
        _fuse_body,
        grid=(grid,),
        in_specs=[
            pl.BlockSpec((2 * _NREL, _BN, _D), lambda i: (0, i, 0)),
            pl.BlockSpec((8, _D), lambda i: (0, 0)),
            pl.BlockSpec(memory_space=pltpu.SMEM),
        ],
        out_specs=[
            pl.BlockSpec((_BN, _D), lambda i: (i, 0)),
            pl.BlockSpec((1, 1, _D), lambda i: (i, 0, 0)),
        ],
        out_shape=[
            jax.ShapeDtypeStruct((_N, _D), jnp.float32),
            jax.ShapeDtypeStruct((grid, 1, _D), jnp.float32),
        ],
    )(partials, rv_pad, relation_bias)


def kernel(h, edge_index_r0, edge_index_r1, edge_index_r2,
           relation_bias, W, node_query, rel_vector):
    q_pad = jnp.zeros((_NREL, 8, _D), jnp.float32).at[:, :_NHEADS].set(node_query)
    rv_pad = jnp.zeros((8, _D), jnp.float32).at[:_NREL].set(rel_vector)

    n_pad = _NCHUNK - _NREAL
    src_all = jnp.stack([edge_index_r0[1], edge_index_r1[1], edge_index_r2[1]])
    src_all = src_all + (jnp.arange(_NREL, dtype=jnp.int32) * _N)[:, None]
    src_pad = jnp.arange(n_pad * _C, dtype=jnp.int32) * 37 % _N
    src_pad = jnp.broadcast_to(src_pad.reshape(1, n_pad, _C),
                               (_NREL, n_pad, _C))
    srcs = jnp.concatenate(
        [src_all.reshape(_NREL, _NREAL, _C), src_pad], axis=1)
    srcs = srcs.reshape(_NREL * _NCHUNK, _C)
    dsts = jnp.stack([edge_index_r0[0], edge_index_r1[0], edge_index_r2[0]])
    dst_pad = _N + jnp.arange(n_pad * _C, dtype=jnp.int32) % _APAD
    dst_pad = jnp.broadcast_to(dst_pad.reshape(1, n_pad, _C),
                               (_NREL, n_pad, _C))
    dsts = jnp.concatenate(
        [dsts.reshape(_NREL, _NREAL, _C), dst_pad], axis=1)
    dsts = dsts.reshape(_NREL * _NCHUNK, _C)
    zeros_hbm = jnp.zeros((_RPT, _D), jnp.float32)

    xbar = _dense_call(h, W, q_pad)
    part = _sc_call(xbar.reshape(_NREL * _N, _D), srcs, dsts, zeros_hbm)
    h_next, attn_p = _fuse_call(part.reshape(2 * _NREL, _N, _D),
                                rv_pad, relation_bias)
    rel_attn_global = attn_p[:, 0, :_NREL].sum(axis=0) * (1.0 / _N)
    return (h_next, rel_attn_global)


# speedup vs baseline: 4.3521x; 1.0037x over previous
import jax
import jax.numpy as jnp
from jax import lax
from jax.experimental import pallas as pl
from jax.experimental.pallas import tpu as pltpu
from jax.experimental.pallas import tpu_sc as plsc

_N = 10000
_E = 320000
_D = 128
_NREL = 3
_NHEADS = 4

_C = 128
_NW = 32
_CPT = 80
_NCHUNK = _NW * _CPT
_NREAL = _E // _C
_RPT = 632
_RPT_LAST = _N - 15 * _RPT
_APAD = 240

_BN = 1000


def _dense_body(h_ref, w_ref, q_ref, out_ref):
    hb = h_ref[...]
    for r in range(_NREL):
        acc = None
        qr = q_ref[r]
        for hd in range(_NHEADS):
            wm = w_ref[r, hd]
            hp = lax.dot_general(hb, wm, (((1,), (1,)), ((), ())),
                                 preferred_element_type=jnp.float32)
            qv = qr[hd:hd + 1, :]
            g = jnp.sum(hp * qv, axis=1, keepdims=True)
            gate = jax.nn.sigmoid(g)
            term = hp * gate
            acc = term if acc is None else acc + term
        out_ref[r] = acc * (1.0 / _NHEADS)


def _dense_call(h, W, q_pad):
    grid = _N // _BN
    return pl.pallas_call(
        _dense_body,
        grid=(grid,),
        in_specs=[
            pl.BlockSpec((_BN, _D), lambda i: (i, 0)),
            pl.BlockSpec((_NREL, _NHEADS, _D, _D), lambda i: (0, 0, 0, 0)),
            pl.BlockSpec((_NREL, 8, _D), lambda i: (0, 0, 0)),
        ],
        out_specs=pl.BlockSpec((_NREL, _BN, _D), lambda i: (0, i, 0)),
        out_shape=jax.ShapeDtypeStruct((_NREL, _N, _D), jnp.float32),
    )(h.astype(jnp.bfloat16), W.astype(jnp.bfloat16), q_pad)


def _sc_body(xbar_hbm, srcs_hbm, dsts_hbm, zeros_hbm, out_hbm, acc,
             si0, si1, si2, si3, di0, di1, di2, di3, rb0, rb1,
             mi0, mi1, mi2, mi3, mg0, mg1, ms0, ms1):
    c = lax.axis_index("c")
    s = lax.axis_index("s")
    wid = s * 2 + c
    base = s * _RPT
    sidx = (si0, si1, si2, si3)
    didx = (di0, di1, di2, di3)
    rows = (rb0, rb1)
    sem_i = (mi0, mi1, mi2, mi3)
    sem_g = (mg0, mg1)
    sem_s = (ms0, ms1)

    def idx_start(cid, k):
        pltpu.async_copy(srcs_hbm.at[cid], sidx[k], sem_i[k])
        pltpu.async_copy(dsts_hbm.at[cid], didx[k], sem_i[k])

    def idx_wait(k):
        pltpu.make_async_copy(srcs_hbm.at[0], sidx[k], sem_i[k]).wait()
        pltpu.make_async_copy(srcs_hbm.at[0], didx[k], sem_i[k]).wait()

    def gather_start(k4, r2):
        pltpu.async_copy(xbar_hbm.at[sidx[k4]], rows[r2], sem_g[r2])

    def gather_wait(r2):
        pltpu.make_async_copy(xbar_hbm.at[pl.ds(0, _C)], rows[r2],
                              sem_g[r2]).wait()

    def scatter_start(r2, k4):
        pltpu.async_copy(rows[r2], acc.at[didx[k4]], sem_s[r2], add=True)

    def scatter_wait(r2):
        pltpu.make_async_copy(xbar_hbm.at[pl.ds(0, _C)], rows[r2],
                              sem_s[r2]).wait()

    def zero_slice():
        @pl.when(s < 15)
        def _():
            pltpu.sync_copy(zeros_hbm, acc.at[pl.ds(base, _RPT)])

        @pl.when(s == 15)
        def _():
            pltpu.sync_copy(zeros_hbm.at[pl.ds(0, _RPT_LAST)],
                            acc.at[pl.ds(base, _RPT_LAST)])

    zero_slice()
    plsc.subcore_barrier()

    for r in range(_NREL):
        tbase = r * _NCHUNK + wid * _CPT
        idx_start(tbase, 0)
        idx_start(tbase + 1, 1)
        idx_start(tbase + 2, 2)
        idx_wait(0)
        gather_start(0, 0)

        def outer(g, carry):
            for b4 in range(4):
                j = g * 4 + b4
                r2 = b4 % 2

                @pl.when(j > 0)
                def _():
                    scatter_wait(1 - r2)

                @pl.when(j < _CPT - 1)
                def _():
                    idx_wait((b4 + 1) % 4)
                    gather_start((b4 + 1) % 4, 1 - r2)

                @pl.when(j < _CPT - 3)
                def _():
                    idx_start(tbase + j + 3, (b4 + 3) % 4)

                gather_wait(r2)
                scatter_start(r2, b4)
            return carry

        lax.fori_loop(0, _CPT // 4, outer, 0)
        scatter_wait(1)
        plsc.subcore_barrier()

        out_off = (c * _NREL + r) * _N + base

        @pl.when(s < 15)
        def _():
            pltpu.sync_copy(acc.at[pl.ds(base, _RPT)],
                            out_hbm.at[pl.ds(out_off, _RPT)])

        @pl.when(s == 15)
        def _():
            pltpu.sync_copy(acc.at[pl.ds(base, _RPT_LAST)],
                            out_hbm.at[pl.ds(out_off, _RPT_LAST)])

        if r < _NREL - 1:
            zero_slice()
        plsc.subcore_barrier()


def _sc_call(xbar_flat, srcs, dsts, zeros_hbm):
    mesh = plsc.VectorSubcoreMesh(core_axis_name="c", subcore_axis_name="s")
    return pl.kernel(
        _sc_body,
        out_type=jax.ShapeDtypeStruct((2 * _NREL * _N, _D), jnp.float32),
        mesh=mesh,
        scratch_types=[
            pltpu.VMEM_SHARED((_N + _APAD, _D), jnp.float32),
            pltpu.VMEM((_C,), jnp.int32),
            pltpu.VMEM((_C,), jnp.int32),
            pltpu.VMEM((_C,), jnp.int32),
            pltpu.VMEM((_C,), jnp.int32),
            pltpu.VMEM((_C,), jnp.int32),
            pltpu.VMEM((_C,), jnp.int32),
            pltpu.VMEM((_C,), jnp.int32),
            pltpu.VMEM((_C,), jnp.int32),
            pltpu.VMEM((_C, _D), jnp.float32),
            pltpu.VMEM((_C, _D), jnp.float32),
            pltpu.SemaphoreType.DMA,
            pltpu.SemaphoreType.DMA,
            pltpu.SemaphoreType.DMA,
            pltpu.SemaphoreType.DMA,
            pltpu.SemaphoreType.DMA,
            pltpu.SemaphoreType.DMA,
            pltpu.SemaphoreType.DMA,
            pltpu.SemaphoreType.DMA,
        ],
    )(xbar_flat, srcs, dsts, zeros_hbm)


def _fuse_body(p_ref, rv_ref, bias_ref, h_ref, attn_ref):
    m0 = p_ref[0] + p_ref[3]
    m1 = p_ref[1] + p_ref[4]
    m2 = p_ref[2] + p_ref[5]
    rv = rv_ref[...]
    l0 = jnp.sum(m0 * rv[0:1, :], axis=1, keepdims=True) + bias_ref[0]
    l1 = jnp.sum(m1 * rv[1:2, :], axis=1, keepdims=True) + bias_ref[1]
    l2 = jnp.sum(m2 * rv[2:3, :], axis=1, keepdims=True) + bias_ref[2]
    mx = jnp.maximum(jnp.maximum(l0, l1), l2)
    e0 = jnp.exp(l0 - mx)
    e1 = jnp.exp(l1 - mx)
    e2 = jnp.exp(l2 - mx)
    inv = 1.0 / (e0 + e1 + e2)
    a0 = e0 * inv
    a1 = e1 * inv
    a2 = e2 * inv
    h_ref[...] = jnp.maximum(a0 * m0 + a1 * m1 + a2 * m2, 0.0)
    col = lax.broadcasted_iota(jnp.int32, (1, _D), 1)
    attn_ref[0] = jnp.where(
        col == 0, jnp.sum(a0),
        jnp.where(col == 1, jnp.sum(a1),
                  jnp.where(col == 2, jnp.sum(a2), 0.0)))


def _fuse_call(partials, rv_pad, relation_bias):
    grid = _N // _BN
    return pl.pallas_call(
        _fuse_body,
        grid=(grid,),
        in_specs=[
            pl.BlockSpec((2 * _NREL, _BN, _D), lambda i: (0, i, 0)),
            pl.BlockSpec((8, _D), lambda i: (0, 0)),
            pl.BlockSpec(memory_space=pltpu.SMEM),
        ],
        out_specs=[
            pl.BlockSpec((_BN, _D), lambda i: (i, 0)),
            pl.BlockSpec((1, 1, _D), lambda i: (i, 0, 0)),
        ],
        out_shape=[
            jax.ShapeDtypeStruct((_N, _D), jnp.float32),
            jax.ShapeDtypeStruct((grid, 1, _D), jnp.float32),
        ],
    )(partials, rv_pad, relation_bias)


def kernel(h, edge_index_r0, edge_index_r1, edge_index_r2,
           relation_bias, W, node_query, rel_vector):
    q_pad = jnp.zeros((_NREL, 8, _D), jnp.float32).at[:, :_NHEADS].set(node_query)
    rv_pad = jnp.zeros((8, _D), jnp.float32).at[:_NREL].set(rel_vector)

    n_pad = _NCHUNK - _NREAL
    src_all = jnp.stack([edge_index_r0[1], edge_index_r1[1], edge_index_r2[1]])
    src_all = src_all + (jnp.arange(_NREL, dtype=jnp.int32) * _N)[:, None]
    src_pad = jnp.arange(n_pad * _C, dtype=jnp.int32) * 37 % _N
    src_pad = jnp.broadcast_to(src_pad.reshape(1, n_pad, _C),
                               (_NREL, n_pad, _C))
    srcs = jnp.concatenate(
        [src_all.reshape(_NREL, _NREAL, _C), src_pad], axis=1)
    srcs = srcs.reshape(_NREL * _NCHUNK, _C)
    dsts = jnp.stack([edge_index_r0[0], edge_index_r1[0], edge_index_r2[0]])
    dst_pad = _N + jnp.arange(n_pad * _C, dtype=jnp.int32) % _APAD
    dst_pad = jnp.broadcast_to(dst_pad.reshape(1, n_pad, _C),
                               (_NREL, n_pad, _C))
    dsts = jnp.concatenate(
        [dsts.reshape(_NREL, _NREAL, _C), dst_pad], axis=1)
    dsts = dsts.reshape(_NREL * _NCHUNK, _C)
    zeros_hbm = jnp.zeros((_RPT, _D), jnp.float32)

    xbar = _dense_call(h, W, q_pad)
    part = _sc_call(xbar.reshape(_NREL * _N, _D), srcs, dsts, zeros_hbm)
    h_next, attn_p = _fuse_call(part.reshape(2 * _NREL, _N, _D),
                                rv_pad, relation_bias)
    rel_attn_global = attn_p[:, 0, :_NREL].sum(axis=0) * (1.0 / _N)
    return (h_next, rel_attn_global)


# speedup vs baseline: 4.3643x; 1.0028x over previous
import jax
import jax.numpy as jnp
from jax import lax
from jax.experimental import pallas as pl
from jax.experimental.pallas import tpu as pltpu
from jax.experimental.pallas import tpu_sc as plsc

_N = 10000
_E = 320000
_D = 128
_NREL = 3
_NHEADS = 4

_C = 128
_NW = 32
_CPT = 80
_NCHUNK = _NW * _CPT
_NREAL = _E // _C
_RPT = 632
_RPT_LAST = _N - 15 * _RPT
_APAD = 240

_BN = 2000


def _dense_body(h_ref, w_ref, q_ref, out_ref):
    hb = h_ref[...]
    for r in range(_NREL):
        acc = None
        qr = q_ref[r]
        for hd in range(_NHEADS):
            wm = w_ref[r, hd]
            hp = lax.dot_general(hb, wm, (((1,), (1,)), ((), ())),
                                 preferred_element_type=jnp.float32)
            qv = qr[hd:hd + 1, :]
            g = jnp.sum(hp * qv, axis=1, keepdims=True)
            gate = jax.nn.sigmoid(g)
            term = hp * gate
            acc = term if acc is None else acc + term
        out_ref[r] = acc * (1.0 / _NHEADS)


def _dense_call(h, W, q_pad):
    grid = _N // _BN
    return pl.pallas_call(
        _dense_body,
        grid=(grid,),
        in_specs=[
            pl.BlockSpec((_BN, _D), lambda i: (i, 0)),
            pl.BlockSpec((_NREL, _NHEADS, _D, _D), lambda i: (0, 0, 0, 0)),
            pl.BlockSpec((_NREL, 8, _D), lambda i: (0, 0, 0)),
        ],
        out_specs=pl.BlockSpec((_NREL, _BN, _D), lambda i: (0, i, 0)),
        out_shape=jax.ShapeDtypeStruct((_NREL, _N, _D), jnp.float32),
    )(h.astype(jnp.bfloat16), W.astype(jnp.bfloat16), q_pad)


def _sc_body(xbar_hbm, srcs_hbm, dsts_hbm, zeros_hbm, out_hbm, acc,
             si0, si1, si2, si3, di0, di1, di2, di3, rb0, rb1,
             mi0, mi1, mi2, mi3, mg0, mg1, ms0, ms1):
    c = lax.axis_index("c")
    s = lax.axis_index("s")
    wid = s * 2 + c
    base = s * _RPT
    sidx = (si0, si1, si2, si3)
    didx = (di0, di1, di2, di3)
    rows = (rb0, rb1)
    sem_i = (mi0, mi1, mi2, mi3)
    sem_g = (mg0, mg1)
    sem_s = (ms0, ms1)

    def idx_start(cid, k):
        pltpu.async_copy(srcs_hbm.at[cid], sidx[k], sem_i[k])
        pltpu.async_copy(dsts_hbm.at[cid], didx[k], sem_i[k])

    def idx_wait(k):
        pltpu.make_async_copy(srcs_hbm.at[0], sidx[k], sem_i[k]).wait()
        pltpu.make_async_copy(srcs_hbm.at[0], didx[k], sem_i[k]).wait()

    def gather_start(k4, r2):
        pltpu.async_copy(xbar_hbm.at[sidx[k4]], rows[r2], sem_g[r2])

    def gather_wait(r2):
        pltpu.make_async_copy(xbar_hbm.at[pl.ds(0, _C)], rows[r2],
                              sem_g[r2]).wait()

    def scatter_start(r2, k4):
        pltpu.async_copy(rows[r2], acc.at[didx[k4]], sem_s[r2], add=True)

    def scatter_wait(r2):
        pltpu.make_async_copy(xbar_hbm.at[pl.ds(0, _C)], rows[r2],
                              sem_s[r2]).wait()

    def zero_slice():
        @pl.when(s < 15)
        def _():
            pltpu.sync_copy(zeros_hbm, acc.at[pl.ds(base, _RPT)])

        @pl.when(s == 15)
        def _():
            pltpu.sync_copy(zeros_hbm.at[pl.ds(0, _RPT_LAST)],
                            acc.at[pl.ds(base, _RPT_LAST)])

    zero_slice()
    plsc.subcore_barrier()

    for r in range(_NREL):
        tbase = r * _NCHUNK + wid * _CPT
        idx_start(tbase, 0)
        idx_start(tbase + 1, 1)
        idx_start(tbase + 2, 2)
        idx_wait(0)
        gather_start(0, 0)

        def outer(g, carry):
            for b4 in range(4):
                j = g * 4 + b4
                r2 = b4 % 2

                @pl.when(j > 0)
                def _():
                    scatter_wait(1 - r2)

                @pl.when(j < _CPT - 1)
                def _():
                    idx_wait((b4 + 1) % 4)
                    gather_start((b4 + 1) % 4, 1 - r2)

                @pl.when(j < _CPT - 3)
                def _():
                    idx_start(tbase + j + 3, (b4 + 3) % 4)

                gather_wait(r2)
                scatter_start(r2, b4)
            return carry

        lax.fori_loop(0, _CPT // 4, outer, 0)
        scatter_wait(1)
        plsc.subcore_barrier()

        out_off = (c * _NREL + r) * _N + base

        @pl.when(s < 15)
        def _():
            pltpu.sync_copy(acc.at[pl.ds(base, _RPT)],
                            out_hbm.at[pl.ds(out_off, _RPT)])

        @pl.when(s == 15)
        def _():
            pltpu.sync_copy(acc.at[pl.ds(base, _RPT_LAST)],
                            out_hbm.at[pl.ds(out_off, _RPT_LAST)])

        if r < _NREL - 1:
            zero_slice()
        plsc.subcore_barrier()


def _sc_call(xbar_flat, srcs, dsts, zeros_hbm):
    mesh = plsc.VectorSubcoreMesh(core_axis_name="c", subcore_axis_name="s")
    return pl.kernel(
        _sc_body,
        out_type=jax.ShapeDtypeStruct((2 * _NREL * _N, _D), jnp.float32),
        mesh=mesh,
        scratch_types=[
            pltpu.VMEM_SHARED((_N + _APAD, _D), jnp.float32),
            pltpu.VMEM((_C,), jnp.int32),
            pltpu.VMEM((_C,), jnp.int32),
            pltpu.VMEM((_C,), jnp.int32),
            pltpu.VMEM((_C,), jnp.int32),
            pltpu.VMEM((_C,), jnp.int32),
            pltpu.VMEM((_C,), jnp.int32),
            pltpu.VMEM((_C,), jnp.int32),
            pltpu.VMEM((_C,), jnp.int32),
            pltpu.VMEM((_C, _D), jnp.float32),
            pltpu.VMEM((_C, _D), jnp.float32),
            pltpu.SemaphoreType.DMA,
            pltpu.SemaphoreType.DMA,
            pltpu.SemaphoreType.DMA,
            pltpu.SemaphoreType.DMA,
            pltpu.SemaphoreType.DMA,
            pltpu.SemaphoreType.DMA,
            pltpu.SemaphoreType.DMA,
            pltpu.SemaphoreType.DMA,
        ],
    )(xbar_flat, srcs, dsts, zeros_hbm)


def _fuse_body(p_ref, rv_ref, bias_ref, h_ref, attn_ref):
    m0 = p_ref[0] + p_ref[3]
    m1 = p_ref[1] + p_ref[4]
    m2 = p_ref[2] + p_ref[5]
    rv = rv_ref[...]
    l0 = jnp.sum(m0 * rv[0:1, :], axis=1, keepdims=True) + bias_ref[0]
    l1 = jnp.sum(m1 * rv[1:2, :], axis=1, keepdims=True) + bias_ref[1]
    l2 = jnp.sum(m2 * rv[2:3, :], axis=1, keepdims=True) + bias_ref[2]
    mx = jnp.maximum(jnp.maximum(l0, l1), l2)
    e0 = jnp.exp(l0 - mx)
    e1 = jnp.exp(l1 - mx)
    e2 = jnp.exp(l2 - mx)
    inv = 1.0 / (e0 + e1 + e2)
    a0 = e0 * inv
    a1 = e1 * inv
    a2 = e2 * inv
    h_ref[...] = jnp.maximum(a0 * m0 + a1 * m1 + a2 * m2, 0.0)
    col = lax.broadcasted_iota(jnp.int32, (1, _D), 1)
    attn_ref[0] = jnp.where(
        col == 0, jnp.sum(a0),
        jnp.where(col == 1, jnp.sum(a1),
                  jnp.where(col == 2, jnp.sum(a2), 0.0)))


def _fuse_call(partials, rv_pad, relation_bias):
    grid = _N // _BN
    return pl.pallas_call(
        _fuse_body,
        grid=(grid,),
        in_specs=[
            pl.BlockSpec((2 * _NREL, _BN, _D), lambda i: (0, i, 0)),
            pl.BlockSpec((8, _D), lambda i: (0, 0)),
            pl.BlockSpec(memory_space=pltpu.SMEM),
        ],
        out_specs=[
            pl.BlockSpec((_BN, _D), lambda i: (i, 0)),
            pl.BlockSpec((1, 1, _D), lambda i: (i, 0, 0)),
        ],
        out_shape=[
            jax.ShapeDtypeStruct((_N, _D), jnp.float32),
            jax.ShapeDtypeStruct((grid, 1, _D), jnp.float32),
        ],
    )(partials, rv_pad, relation_bias)


def kernel(h, edge_index_r0, edge_index_r1, edge_index_r2,
           relation_bias, W, node_query, rel_vector):
    q_pad = jnp.zeros((_NREL, 8, _D), jnp.float32).at[:, :_NHEADS].set(node_query)
    rv_pad = jnp.zeros((8, _D), jnp.float32).at[:_NREL].set(rel_vector)

    n_pad = _NCHUNK - _NREAL
    src_all = jnp.stack([edge_index_r0[1], edge_index_r1[1], edge_index_r2[1]])
    src_all = src_all + (jnp.arange(_NREL, dtype=jnp.int32) * _N)[:, None]
    src_pad = jnp.arange(n_pad * _C, dtype=jnp.int32) * 37 % _N
    src_pad = jnp.broadcast_to(src_pad.reshape(1, n_pad, _C),
                               (_NREL, n_pad, _C))
    srcs = jnp.concatenate(
        [src_all.reshape(_NREL, _NREAL, _C), src_pad], axis=1)
    srcs = srcs.reshape(_NREL * _NCHUNK, _C)
    dsts = jnp.stack([edge_index_r0[0], edge_index_r1[0], edge_index_r2[0]])
    dst_pad = _N + jnp.arange(n_pad * _C, dtype=jnp.int32) % _APAD
    dst_pad = jnp.broadcast_to(dst_pad.reshape(1, n_pad, _C),
                               (_NREL, n_pad, _C))
    dsts = jnp.concatenate(
        [dsts.reshape(_NREL, _NREAL, _C), dst_pad], axis=1)
    dsts = dsts.reshape(_NREL * _NCHUNK, _C)
    zeros_hbm = jnp.zeros((_RPT, _D), jnp.float32)

    xbar = _dense_call(h, W, q_pad)
    part = _sc_call(xbar.reshape(_NREL * _N, _D), srcs, dsts, zeros_hbm)
    h_next, attn_p = _fuse_call(part.reshape(2 * _NREL, _N, _D),
                                rv_pad, relation_bias)
    rel_attn_global = attn_p[:, 0, :_NREL].sum(axis=0) * (1.0 / _N)
    return (h_next, rel_attn_global)
